# Initial kernel scaffold; baseline (speedup 1.0000x reference)
import jax
import jax.numpy as jnp
from jax.experimental import pallas as pl


def kernel(x, orbitals_mf, orbitals_hf):
    raise NotImplementedError("write your pallas kernel here")


# speedup vs baseline: 2.1975x; 2.1975x over previous
import functools

import jax
import jax.numpy as jnp
from jax import lax
from jax.experimental import pallas as pl
from jax.experimental.pallas import tpu as pltpu
from jax.experimental.pallas import tpu_sc as plsc

L = 16
NW = 32
N_SAMPLES = 4096
N_SITES = 100
D = 128
SITES_PAD = 112
N_CHUNKS = SITES_PAD // L
SPW = N_SAMPLES // NW
ROWS_PW = SPW * N_SITES
GCHUNK = 128
N_GCHUNKS = ROWS_PW // GCHUNK


def _vfull(val):
    return jnp.full((L,), val, jnp.int32)


def _bcast_last(cs):
    return plsc.cummax(lax.rev(cs, (0,)))


def _sc_body(x_hbm, table_hbm, out_hbm, x_v, idx2d, rows_a, rows_b, gsem):
    wid = lax.axis_index("s") * 2 + lax.axis_index("c")
    base_s = wid * SPW

    pltpu.sync_copy(x_hbm.at[pl.ds(base_s, SPW)], x_v)

    iota = lax.iota(jnp.int32, L)
    ones_v = _vfull(1)
    zeros_v = _vfull(0)
    negones_v = _vfull(-1)

    def build_sample(smp, smp_off):
        n_up = zeros_v
        for c in range(N_CHUNKS):
            v = x_v[smp, pl.ds(c * L, L)]
            upi = jnp.where(v == ones_v, ones_v, zeros_v)
            n_up = n_up + _bcast_last(plsc.cumsum(upi))
        carry = zeros_v
        for c in range(N_CHUNKS):
            v = x_v[smp, pl.ds(c * L, L)]
            up = v == ones_v
            dn = v == negones_v
            upi = jnp.where(up, ones_v, zeros_v)
            cs = plsc.cumsum(upi)
            up_ex = carry + cs - upi
            i_loc = iota + _vfull(c * L)
            p = jnp.where(up, up_ex, n_up + i_loc - up_ex)
            src = jnp.where(dn, i_loc + _vfull(N_SITES), i_loc)
            dest = smp_off + p
            row = lax.shift_right_logical(dest, _vfull(7))
            col = dest & _vfull(GCHUNK - 1)
            plsc.store_scatter(idx2d, [row, col], src, mask=up | dn)
            carry = carry + _bcast_last(cs)
        return smp_off + _vfull(N_SITES)

    lax.fori_loop(0, SPW, build_sample, zeros_v)

    out_base = wid * ROWS_PW

    def gather(k, buf):
        return pltpu.async_copy(table_hbm.at[idx2d.at[k]], buf, gsem)

    gather(0, rows_a).wait()

    def move(k, _):
        even = (k & 1) == 0

        @pl.when(even)
        def _():
            @pl.when(k + 1 < N_GCHUNKS)
            def _():
                gather(k + 1, rows_b).wait()
            pltpu.sync_copy(rows_a, out_hbm.at[pl.ds(out_base + k * GCHUNK, GCHUNK)])

        @pl.when(jnp.logical_not(even))
        def _():
            @pl.when(k + 1 < N_GCHUNKS)
            def _():
                gather(k + 1, rows_a).wait()
            pltpu.sync_copy(rows_b, out_hbm.at[pl.ds(out_base + k * GCHUNK, GCHUNK)])

        return 0

    lax.fori_loop(0, N_GCHUNKS, move, 0)


_sc_kernel = functools.partial(
    pl.kernel,
    out_type=jax.ShapeDtypeStruct((N_SAMPLES * N_SITES, D), jnp.float32),
    mesh=plsc.VectorSubcoreMesh(core_axis_name="c", subcore_axis_name="s"),
    compiler_params=pltpu.CompilerParams(needs_layout_passes=False),
    scratch_types=[
        pltpu.VMEM((SPW, SITES_PAD), jnp.int32),
        pltpu.VMEM((N_GCHUNKS, GCHUNK), jnp.int32),
        pltpu.VMEM((GCHUNK, D), jnp.float32),
        pltpu.VMEM((GCHUNK, D), jnp.float32),
        pltpu.SemaphoreType.DMA,
    ],
)(_sc_body)


def kernel(x, orbitals_mf, orbitals_hf):
    n_samples, n_sites = x.shape
    assert (n_samples, n_sites) == (N_SAMPLES, N_SITES)
    table = jnp.concatenate([orbitals_mf, orbitals_hf], axis=1)
    xp = jnp.pad(x.astype(jnp.int32), ((0, 0), (0, SITES_PAD - n_sites)))
    out = _sc_kernel(xp, table)
    return out.reshape(n_samples, n_sites, D)
